# baseline (speedup 1.0000x reference)
import functools

import jax
import jax.numpy as jnp
from jax import lax
from jax.experimental import pallas as pl
from jax.experimental.pallas import tpu as pltpu
from jax.experimental.pallas import tpu_sc as plsc

N = 10000
K = 32
LATENT = 512
SCALE_GRAD = 0.4 / N
SCALE_E = 0.1 / N

NW = 32
VPW0 = 320
VPW1 = 320
BASE1 = 16 * VPW0
NPAD = 16 * VPW0 + 16 * VPW1
NBLK0 = VPW0 // 16
NBLK1 = VPW1 // 16
TAILV = N - (BASE1 + 15 * VPW1)
M = 3 * N
MPAD = 3 * NPAD
SLAB = K * VPW0
NK = N * K
TILE = 2048
GRID = MPAD // TILE

_mesh = plsc.VectorSubcoreMesh(core_axis_name="c", subcore_axis_name="s")
_sc_params = pltpu.CompilerParams(needs_layout_passes=False)


def _fwd_body(code_ref, wt_ref, b_ref, out_ref):
    t = pl.program_id(0)
    r = lax.dot_general(code_ref[...], wt_ref[...], (((1,), (1,)), ((), ())),
                        preferred_element_type=jnp.float32)
    r = r + b_ref[...]
    col = t * TILE + lax.broadcasted_iota(jnp.int32, (1, TILE), 1)
    out_ref[...] = jnp.where(col < M, r, 0.0)


_fwd_call = pl.pallas_call(
    _fwd_body,
    grid=(GRID,),
    in_specs=[
        pl.BlockSpec((1, LATENT), lambda t: (0, 0)),
        pl.BlockSpec((TILE, LATENT), lambda t: (t, 0)),
        pl.BlockSpec((1, TILE), lambda t: (0, t)),
    ],
    out_specs=pl.BlockSpec((1, TILE), lambda t: (0, t)),
    out_shape=jax.ShapeDtypeStruct((1, MPAD), jnp.float32),
)


def _copy_slabs(srcs_dsts, c, s, sem):
    def _fire(base, vpw_src, vpw_dst):
        handles = []
        for src_hbm, dst_v in srcs_dsts:
            for k in range(K):
                handles.append(pltpu.async_copy(
                    src_hbm.at[pl.ds(k * N + base, vpw_src)],
                    dst_v.at[pl.ds(k * vpw_dst, vpw_src)], sem))
        for h in handles:
            h.wait()

    @pl.when(c == 0)
    def _():
        _fire(s * VPW0, VPW0, VPW0)

    @pl.when(jnp.logical_and(c == 1, s != 15))
    def _():
        _fire(BASE1 + s * VPW1, VPW1, VPW1)

    @pl.when(jnp.logical_and(c == 1, s == 15))
    def _():
        _fire(BASE1 + 15 * VPW1, TAILV, VPW1)


@functools.partial(
    pl.kernel,
    out_type=[
        jax.ShapeDtypeStruct((NW, SLAB), jnp.float32),
        jax.ShapeDtypeStruct((NW, SLAB), jnp.float32),
        jax.ShapeDtypeStruct((NW, SLAB), jnp.int32),
    ],
    mesh=_mesh,
    scratch_types=[
        pltpu.VMEM((MPAD,), jnp.float32),
        pltpu.VMEM((SLAB,), jnp.int32),
        pltpu.VMEM((SLAB,), jnp.float32),
        pltpu.VMEM((VPW0,), jnp.int32),
        pltpu.VMEM((VPW0,), jnp.float32),
        pltpu.VMEM((SLAB,), jnp.float32),
        pltpu.VMEM((SLAB,), jnp.float32),
        pltpu.SemaphoreType.DMA,
    ],
    compiler_params=_sc_params,
)
def _pre_call(xyz_hbm, nbr_hbm, w_hbm, nn_hbm, area_hbm,
              s0_hbm, wo_hbm, nbro_hbm,
              xyz_v, nbr_v, w_v, nn_v, area_v, s0_v, wf_v, sem):
    c = lax.axis_index("c")
    s = lax.axis_index("s")
    wid = s * 2 + c
    vpw = jnp.where(c == 0, VPW0, VPW1)
    base = jnp.where(c == 0, s * VPW0, BASE1 + s * VPW1)
    is_tail = jnp.logical_and(c == 1, s == 15)
    nblk = jnp.where(c == 0, NBLK0, jnp.where(is_tail, TAILV // 16, NBLK1))

    zeros16 = jnp.zeros((16,), jnp.float32)
    izeros16 = jnp.zeros((16,), jnp.int32)

    xyz_cp = pltpu.async_copy(xyz_hbm, xyz_v.at[pl.ds(0, M)], sem)
    _copy_slabs([(nbr_hbm, nbr_v), (w_hbm, w_v)], c, s, sem)
    xyz_cp.wait()
    for u in range((MPAD - M) // 16):
        xyz_v[pl.ds(M + u * 16, 16)] = zeros16

    @pl.when(c == 0)
    def _():
        pltpu.sync_copy(nn_hbm.at[pl.ds(s * VPW0, VPW0)],
                        nn_v.at[pl.ds(0, VPW0)])
        pltpu.sync_copy(area_hbm.at[pl.ds(s * VPW0, VPW0)],
                        area_v.at[pl.ds(0, VPW0)])

    @pl.when(jnp.logical_and(c == 1, s != 15))
    def _():
        pltpu.sync_copy(nn_hbm.at[pl.ds(BASE1 + s * VPW1, VPW1)],
                        nn_v.at[pl.ds(0, VPW1)])
        pltpu.sync_copy(area_hbm.at[pl.ds(BASE1 + s * VPW1, VPW1)],
                        area_v.at[pl.ds(0, VPW1)])

    @pl.when(jnp.logical_and(c == 1, s == 15))
    def _():
        pltpu.sync_copy(nn_hbm.at[pl.ds(BASE1 + 15 * VPW1, TAILV)],
                        nn_v.at[pl.ds(0, TAILV)])
        pltpu.sync_copy(area_hbm.at[pl.ds(BASE1 + 15 * VPW1, TAILV)],
                        area_v.at[pl.ds(0, TAILV)])

    def _block(b):
        v0 = b * 16
        g0 = base + v0
        px = xyz_v[pl.ds(g0, 16)]
        py = xyz_v[pl.ds(N + g0, 16)]
        pz = xyz_v[pl.ds(2 * N + g0, 16)]
        nnv = nn_v[pl.ds(v0, 16)]
        areav = area_v[pl.ds(v0, 16)]
        for k in range(K):
            o = k * vpw + v0
            nbr = nbr_v[pl.ds(o, 16)]
            w = w_v[pl.ds(o, 16)]
            wf = jnp.where(nnv > k, w * areav, 0.0)
            qx = plsc.load_gather(xyz_v, [nbr])
            qy = plsc.load_gather(xyz_v, [nbr + N])
            qz = plsc.load_gather(xyz_v, [nbr + 2 * N])
            dx = px - qx
            dy = py - qy
            dz = pz - qz
            s0_v[pl.ds(o, 16)] = dx * dx + dy * dy + dz * dz
            wf_v[pl.ds(o, 16)] = wf

    plsc.parallel_loop(0, nblk, 1, unroll=4)(_block)
    pltpu.sync_copy(s0_v, s0_hbm.at[wid])
    pltpu.sync_copy(wf_v, wo_hbm.at[wid])
    pltpu.sync_copy(nbr_v, nbro_hbm.at[wid])


@functools.partial(
    pl.kernel,
    out_type=[
        jax.ShapeDtypeStruct((NW, MPAD), jnp.float32),
        jax.ShapeDtypeStruct((NW, 16), jnp.float32),
    ],
    mesh=_mesh,
    scratch_types=[
        pltpu.VMEM((MPAD,), jnp.float32),
        pltpu.VMEM((MPAD,), jnp.float32),
        pltpu.VMEM((SLAB,), jnp.float32),
        pltpu.VMEM((SLAB,), jnp.int32),
        pltpu.VMEM((SLAB,), jnp.float32),
        pltpu.VMEM((16,), jnp.float32),
        pltpu.SemaphoreType.DMA,
    ],
    compiler_params=_sc_params,
)
def _edge_call(recon_hbm, nbr_hbm, s0_hbm, w_hbm,
               gpart_hbm, epart_hbm,
               recon_v, grad_v, s0_v, nbr_v, w_v, e_v, sem):
    c = lax.axis_index("c")
    s = lax.axis_index("s")
    wid = s * 2 + c
    vpw = jnp.where(c == 0, VPW0, VPW1)
    base = jnp.where(c == 0, s * VPW0, BASE1 + s * VPW1)
    is_tail = jnp.logical_and(c == 1, s == 15)
    nblk = jnp.where(c == 0, NBLK0, jnp.where(is_tail, TAILV // 16, NBLK1))

    zeros16 = jnp.zeros((16,), jnp.float32)

    cps = [pltpu.async_copy(recon_hbm, recon_v, sem),
           pltpu.async_copy(s0_hbm.at[wid], s0_v, sem),
           pltpu.async_copy(w_hbm.at[wid], w_v, sem),
           pltpu.async_copy(nbr_hbm.at[wid], nbr_v, sem)]

    def _zero(z, _):
        b = z * 256
        for u in range(16):
            grad_v[pl.ds(b + u * 16, 16)] = zeros16
        return 0

    lax.fori_loop(0, MPAD // 256, _zero, 0)
    for h in cps:
        h.wait()

    iota16 = lax.iota(jnp.int32, 16)

    def _block(b, eacc):
        v0 = b * 16
        g0 = base + v0
        sidx = 3 * g0 + 3 * iota16
        sx = plsc.load_gather(recon_v, [sidx])
        sy = plsc.load_gather(recon_v, [sidx + 1])
        sz = plsc.load_gather(recon_v, [sidx + 2])

        gx = zeros16
        gy = zeros16
        gz = zeros16
        ek = zeros16
        for k in range(K):
            o = k * vpw + v0
            nbr = nbr_v[pl.ds(o, 16)]
            wf = w_v[pl.ds(o, 16)]
            s0 = s0_v[pl.ds(o, 16)]
            jb = nbr * 3
            rx = plsc.load_gather(recon_v, [jb])
            ry = plsc.load_gather(recon_v, [jb + 1])
            rz = plsc.load_gather(recon_v, [jb + 2])
            e1x = sx - rx
            e1y = sy - ry
            e1z = sz - rz
            d = (e1x * e1x + e1y * e1y + e1z * e1z) - s0
            wmd = wf * d
            ek = ek + wmd * d
            q = wmd * SCALE_GRAD
            cx = q * e1x
            cy = q * e1y
            cz = q * e1z
            gx = gx + cx
            gy = gy + cy
            gz = gz + cz
            plsc.addupdate_scatter(grad_v, [jb], -cx)
            plsc.addupdate_scatter(grad_v, [jb + 1], -cy)
            plsc.addupdate_scatter(grad_v, [jb + 2], -cz)

        plsc.addupdate_scatter(grad_v, [sidx], gx)
        plsc.addupdate_scatter(grad_v, [sidx + 1], gy)
        plsc.addupdate_scatter(grad_v, [sidx + 2], gz)
        return eacc + ek

    eacc = plsc.parallel_loop(0, nblk, 1, unroll=4, carry=zeros16)(_block)
    e_v[...] = eacc
    pltpu.sync_copy(grad_v, gpart_hbm.at[wid])
    pltpu.sync_copy(e_v, epart_hbm.at[wid])


def _bwd_body(wt_ref, gp_ref, ep_ref, gc_ref, e_ref):
    t = pl.program_id(0)

    @pl.when(t == 0)
    def _():
        gc_ref[...] = jnp.zeros_like(gc_ref)
        e_ref[...] = (jnp.sum(ep_ref[...]) * SCALE_E).reshape(1, 1)

    row = t * TILE + lax.broadcasted_iota(jnp.int32, (TILE, 1), 0)
    wm = jnp.where(row < M, wt_ref[...], 0.0)
    g = jnp.sum(gp_ref[...], axis=0, keepdims=True)
    contrib = lax.dot_general(g, wm, (((1,), (0,)), ((), ())),
                              preferred_element_type=jnp.float32)
    gc_ref[...] += contrib


_bwd_call = pl.pallas_call(
    _bwd_body,
    grid=(GRID,),
    in_specs=[
        pl.BlockSpec((TILE, LATENT), lambda t: (t, 0)),
        pl.BlockSpec((NW, TILE), lambda t: (0, t)),
        pl.BlockSpec((NW, 16), lambda t: (0, 0)),
    ],
    out_specs=[
        pl.BlockSpec((1, LATENT), lambda t: (0, 0)),
        pl.BlockSpec((1, 1), lambda t: (0, 0)),
    ],
    out_shape=[
        jax.ShapeDtypeStruct((1, LATENT), jnp.float32),
        jax.ShapeDtypeStruct((1, 1), jnp.float32),
    ],
)


def kernel(code, W_dec, b_dec, xyz1, neighbors, num_neighbors, weights, area):
    Wt = W_dec.T
    xyzP = xyz1.T.reshape(M)
    nbrKF = neighbors.astype(jnp.int32).T.reshape(NK)
    wKF = weights.T.reshape(NK)
    nnI = num_neighbors.astype(jnp.int32)

    s0, wS, nbrS = _pre_call(xyzP, nbrKF, wKF, nnI, area)

    b_pad = jnp.pad(b_dec, (0, MPAD - M)).reshape(1, MPAD)
    recon = _fwd_call(code.reshape(1, LATENT), Wt, b_pad).reshape(MPAD)

    gpart, epart = _edge_call(recon, nbrS, s0, wS)

    gc, e = _bwd_call(Wt, gpart, epart)
    return e[0, 0], gc[0]


# speedup vs baseline: 1.0344x; 1.0344x over previous
import functools

import jax
import jax.numpy as jnp
from jax import lax
from jax.experimental import pallas as pl
from jax.experimental.pallas import tpu as pltpu
from jax.experimental.pallas import tpu_sc as plsc

N = 10000
K = 32
LATENT = 512
SCALE_GRAD = 0.4 / N
SCALE_E = 0.1 / N

NW = 32
VPW0 = 320
VPW1 = 320
BASE1 = 16 * VPW0
NPAD = 16 * VPW0 + 16 * VPW1
NBLK0 = VPW0 // 16
NBLK1 = VPW1 // 16
TAILV = N - (BASE1 + 15 * VPW1)
M = 3 * N
MPAD = 3 * NPAD
SLAB = K * VPW0
NK = N * K
TILE = 2048
GRID = MPAD // TILE

_mesh = plsc.VectorSubcoreMesh(core_axis_name="c", subcore_axis_name="s")
_sc_params = pltpu.CompilerParams(needs_layout_passes=False)


def _fwd_body(code_ref, wt_ref, b_ref, out_ref):
    t = pl.program_id(0)
    r = lax.dot_general(code_ref[...], wt_ref[...], (((1,), (1,)), ((), ())),
                        preferred_element_type=jnp.float32)
    r = r + b_ref[...]
    col = t * TILE + lax.broadcasted_iota(jnp.int32, (1, TILE), 1)
    out_ref[...] = jnp.where(col < M, r, 0.0)


_fwd_call = pl.pallas_call(
    _fwd_body,
    grid=(GRID,),
    in_specs=[
        pl.BlockSpec((1, LATENT), lambda t: (0, 0)),
        pl.BlockSpec((TILE, LATENT), lambda t: (t, 0)),
        pl.BlockSpec((1, TILE), lambda t: (0, t)),
    ],
    out_specs=pl.BlockSpec((1, TILE), lambda t: (0, t)),
    out_shape=jax.ShapeDtypeStruct((1, MPAD), jnp.float32),
)


def _copy_slabs(srcs_dsts, c, s, sem):
    def _fire(base, vpw_src, vpw_dst):
        handles = []
        for src_hbm, dst_v in srcs_dsts:
            for k in range(K):
                handles.append(pltpu.async_copy(
                    src_hbm.at[pl.ds(k * N + base, vpw_src)],
                    dst_v.at[pl.ds(k * vpw_dst, vpw_src)], sem))
        for h in handles:
            h.wait()

    @pl.when(c == 0)
    def _():
        _fire(s * VPW0, VPW0, VPW0)

    @pl.when(jnp.logical_and(c == 1, s != 15))
    def _():
        _fire(BASE1 + s * VPW1, VPW1, VPW1)

    @pl.when(jnp.logical_and(c == 1, s == 15))
    def _():
        _fire(BASE1 + 15 * VPW1, TAILV, VPW1)


@functools.partial(
    pl.kernel,
    out_type=[
        jax.ShapeDtypeStruct((NW, SLAB), jnp.float32),
        jax.ShapeDtypeStruct((NW, SLAB), jnp.float32),
        jax.ShapeDtypeStruct((NW, SLAB), jnp.int32),
    ],
    mesh=_mesh,
    scratch_types=[
        pltpu.VMEM((MPAD,), jnp.float32),
        pltpu.VMEM((SLAB,), jnp.int32),
        pltpu.VMEM((SLAB,), jnp.float32),
        pltpu.VMEM((VPW0,), jnp.int32),
        pltpu.VMEM((VPW0,), jnp.float32),
        pltpu.VMEM((SLAB,), jnp.float32),
        pltpu.VMEM((SLAB,), jnp.float32),
        pltpu.SemaphoreType.DMA,
    ],
    compiler_params=_sc_params,
)
def _pre_call(xyz_hbm, nbr_hbm, w_hbm, nn_hbm, area_hbm,
              s0_hbm, wo_hbm, nbro_hbm,
              xyz_v, nbr_v, w_v, nn_v, area_v, s0_v, wf_v, sem):
    c = lax.axis_index("c")
    s = lax.axis_index("s")
    wid = s * 2 + c
    vpw = jnp.where(c == 0, VPW0, VPW1)
    base = jnp.where(c == 0, s * VPW0, BASE1 + s * VPW1)
    is_tail = jnp.logical_and(c == 1, s == 15)
    nblk = jnp.where(c == 0, NBLK0, jnp.where(is_tail, TAILV // 16, NBLK1))

    zeros16 = jnp.zeros((16,), jnp.float32)
    izeros16 = jnp.zeros((16,), jnp.int32)

    xyz_cp = pltpu.async_copy(xyz_hbm, xyz_v.at[pl.ds(0, M)], sem)
    _copy_slabs([(nbr_hbm, nbr_v), (w_hbm, w_v)], c, s, sem)
    xyz_cp.wait()
    for u in range((MPAD - M) // 16):
        xyz_v[pl.ds(M + u * 16, 16)] = zeros16

    @pl.when(c == 0)
    def _():
        pltpu.sync_copy(nn_hbm.at[pl.ds(s * VPW0, VPW0)],
                        nn_v.at[pl.ds(0, VPW0)])
        pltpu.sync_copy(area_hbm.at[pl.ds(s * VPW0, VPW0)],
                        area_v.at[pl.ds(0, VPW0)])

    @pl.when(jnp.logical_and(c == 1, s != 15))
    def _():
        pltpu.sync_copy(nn_hbm.at[pl.ds(BASE1 + s * VPW1, VPW1)],
                        nn_v.at[pl.ds(0, VPW1)])
        pltpu.sync_copy(area_hbm.at[pl.ds(BASE1 + s * VPW1, VPW1)],
                        area_v.at[pl.ds(0, VPW1)])

    @pl.when(jnp.logical_and(c == 1, s == 15))
    def _():
        pltpu.sync_copy(nn_hbm.at[pl.ds(BASE1 + 15 * VPW1, TAILV)],
                        nn_v.at[pl.ds(0, TAILV)])
        pltpu.sync_copy(area_hbm.at[pl.ds(BASE1 + 15 * VPW1, TAILV)],
                        area_v.at[pl.ds(0, TAILV)])

    def _block(b):
        v0 = b * 16
        g0 = base + v0
        px = xyz_v[pl.ds(g0, 16)]
        py = xyz_v[pl.ds(N + g0, 16)]
        pz = xyz_v[pl.ds(2 * N + g0, 16)]
        nnv = nn_v[pl.ds(v0, 16)]
        areav = area_v[pl.ds(v0, 16)]
        for k in range(K):
            o = k * vpw + v0
            nbr = nbr_v[pl.ds(o, 16)]
            w = w_v[pl.ds(o, 16)]
            wf = jnp.where(nnv > k, w * areav, 0.0)
            qx = plsc.load_gather(xyz_v, [nbr])
            qy = plsc.load_gather(xyz_v, [nbr + N])
            qz = plsc.load_gather(xyz_v, [nbr + 2 * N])
            dx = px - qx
            dy = py - qy
            dz = pz - qz
            s0_v[pl.ds(o, 16)] = dx * dx + dy * dy + dz * dz
            wf_v[pl.ds(o, 16)] = wf

    plsc.parallel_loop(0, nblk, 1, unroll=2)(_block)
    pltpu.sync_copy(s0_v, s0_hbm.at[wid])
    pltpu.sync_copy(wf_v, wo_hbm.at[wid])
    pltpu.sync_copy(nbr_v, nbro_hbm.at[wid])


@functools.partial(
    pl.kernel,
    out_type=[
        jax.ShapeDtypeStruct((NW, MPAD), jnp.float32),
        jax.ShapeDtypeStruct((NW, 16), jnp.float32),
    ],
    mesh=_mesh,
    scratch_types=[
        pltpu.VMEM((MPAD,), jnp.float32),
        pltpu.VMEM((MPAD,), jnp.float32),
        pltpu.VMEM((SLAB,), jnp.float32),
        pltpu.VMEM((SLAB,), jnp.int32),
        pltpu.VMEM((SLAB,), jnp.float32),
        pltpu.VMEM((16,), jnp.float32),
        pltpu.SemaphoreType.DMA,
    ],
    compiler_params=_sc_params,
)
def _edge_call(recon_hbm, nbr_hbm, s0_hbm, w_hbm,
               gpart_hbm, epart_hbm,
               recon_v, grad_v, s0_v, nbr_v, w_v, e_v, sem):
    c = lax.axis_index("c")
    s = lax.axis_index("s")
    wid = s * 2 + c
    vpw = jnp.where(c == 0, VPW0, VPW1)
    base = jnp.where(c == 0, s * VPW0, BASE1 + s * VPW1)
    is_tail = jnp.logical_and(c == 1, s == 15)
    nblk = jnp.where(c == 0, NBLK0, jnp.where(is_tail, TAILV // 16, NBLK1))

    zeros16 = jnp.zeros((16,), jnp.float32)

    cps = [pltpu.async_copy(recon_hbm, recon_v, sem),
           pltpu.async_copy(s0_hbm.at[wid], s0_v, sem),
           pltpu.async_copy(w_hbm.at[wid], w_v, sem),
           pltpu.async_copy(nbr_hbm.at[wid], nbr_v, sem)]

    def _zero(z, _):
        b = z * 256
        for u in range(16):
            grad_v[pl.ds(b + u * 16, 16)] = zeros16
        return 0

    lax.fori_loop(0, MPAD // 256, _zero, 0)
    for h in cps:
        h.wait()

    iota16 = lax.iota(jnp.int32, 16)

    def _block(b, eacc):
        v0 = b * 16
        g0 = base + v0
        sidx = 3 * g0 + 3 * iota16
        sx = plsc.load_gather(recon_v, [sidx])
        sy = plsc.load_gather(recon_v, [sidx + 1])
        sz = plsc.load_gather(recon_v, [sidx + 2])

        gx = zeros16
        gy = zeros16
        gz = zeros16
        ek = zeros16
        for k in range(K):
            o = k * vpw + v0
            nbr = nbr_v[pl.ds(o, 16)]
            wf = w_v[pl.ds(o, 16)]
            s0 = s0_v[pl.ds(o, 16)]
            jb = nbr * 3
            rx = plsc.load_gather(recon_v, [jb])
            ry = plsc.load_gather(recon_v, [jb + 1])
            rz = plsc.load_gather(recon_v, [jb + 2])
            e1x = sx - rx
            e1y = sy - ry
            e1z = sz - rz
            d = (e1x * e1x + e1y * e1y + e1z * e1z) - s0
            wmd = wf * d
            ek = ek + wmd * d
            q = wmd * SCALE_GRAD
            cx = q * e1x
            cy = q * e1y
            cz = q * e1z
            gx = gx + cx
            gy = gy + cy
            gz = gz + cz
            plsc.addupdate_scatter(grad_v, [jb], -cx)
            plsc.addupdate_scatter(grad_v, [jb + 1], -cy)
            plsc.addupdate_scatter(grad_v, [jb + 2], -cz)

        plsc.addupdate_scatter(grad_v, [sidx], gx)
        plsc.addupdate_scatter(grad_v, [sidx + 1], gy)
        plsc.addupdate_scatter(grad_v, [sidx + 2], gz)
        return eacc + ek

    eacc = plsc.parallel_loop(0, nblk, 1, unroll=2, carry=zeros16)(_block)
    e_v[...] = eacc
    pltpu.sync_copy(grad_v, gpart_hbm.at[wid])
    pltpu.sync_copy(e_v, epart_hbm.at[wid])


def _bwd_body(wt_ref, gp_ref, ep_ref, gc_ref, e_ref):
    t = pl.program_id(0)

    @pl.when(t == 0)
    def _():
        gc_ref[...] = jnp.zeros_like(gc_ref)
        e_ref[...] = (jnp.sum(ep_ref[...]) * SCALE_E).reshape(1, 1)

    row = t * TILE + lax.broadcasted_iota(jnp.int32, (TILE, 1), 0)
    wm = jnp.where(row < M, wt_ref[...], 0.0)
    g = jnp.sum(gp_ref[...], axis=0, keepdims=True)
    contrib = lax.dot_general(g, wm, (((1,), (0,)), ((), ())),
                              preferred_element_type=jnp.float32)
    gc_ref[...] += contrib


_bwd_call = pl.pallas_call(
    _bwd_body,
    grid=(GRID,),
    in_specs=[
        pl.BlockSpec((TILE, LATENT), lambda t: (t, 0)),
        pl.BlockSpec((NW, TILE), lambda t: (0, t)),
        pl.BlockSpec((NW, 16), lambda t: (0, 0)),
    ],
    out_specs=[
        pl.BlockSpec((1, LATENT), lambda t: (0, 0)),
        pl.BlockSpec((1, 1), lambda t: (0, 0)),
    ],
    out_shape=[
        jax.ShapeDtypeStruct((1, LATENT), jnp.float32),
        jax.ShapeDtypeStruct((1, 1), jnp.float32),
    ],
)


def kernel(code, W_dec, b_dec, xyz1, neighbors, num_neighbors, weights, area):
    Wt = W_dec.T
    xyzP = xyz1.T.reshape(M)
    nbrKF = neighbors.astype(jnp.int32).T.reshape(NK)
    wKF = weights.T.reshape(NK)
    nnI = num_neighbors.astype(jnp.int32)

    s0, wS, nbrS = _pre_call(xyzP, nbrKF, wKF, nnI, area)

    b_pad = jnp.pad(b_dec, (0, MPAD - M)).reshape(1, MPAD)
    recon = _fwd_call(code.reshape(1, LATENT), Wt, b_pad).reshape(MPAD)

    gpart, epart = _edge_call(recon, nbrS, s0, wS)

    gc, e = _bwd_call(Wt, gpart, epart)
    return e[0, 0], gc[0]
